# Initial kernel scaffold; baseline (speedup 1.0000x reference)
import jax
import jax.numpy as jnp
from jax.experimental import pallas as pl


def kernel(x_nchw, w1, gamma, beta, w2):
    raise NotImplementedError("write your pallas kernel here")


# speedup vs baseline: 2.5230x; 2.5230x over previous
import functools

import jax
import jax.numpy as jnp
from jax.experimental import pallas as pl
from jax.experimental.pallas import tpu as pltpu

_BN_EPS = 1e-5


def _round_up(v, m):
    return ((v + m - 1) // m) * m


def _build_im2col(x_ref, ext_ref, im2_ref, *, W, P, pad):
    cin = x_ref.shape[1]
    ext_ref[:, :pad] = jnp.zeros((cin, pad), jnp.bfloat16)
    ext_ref[:, pad + P:] = jnp.zeros((cin, pad), jnp.bfloat16)
    ext_ref[:, pad:pad + P] = x_ref[0].astype(jnp.bfloat16)

    pix = jax.lax.broadcasted_iota(jnp.int32, (1, P), 1)
    if (W & (W - 1)) == 0:
        col = jnp.bitwise_and(pix, W - 1)
    else:
        col = jax.lax.rem(pix, W)
    not_left = col > 0
    not_right = col < (W - 1)
    col_masks = (not_left, None, not_right)

    for dy in range(3):
        for dx in range(3):
            off = (dy - 1) * W + (dx - 1)
            slab = ext_ref[:, pad + off: pad + off + P]
            mask = col_masks[dx]
            if mask is not None:
                slab = jnp.where(mask, slab, jnp.bfloat16(0))
            t = dy * 3 + dx
            im2_ref[t * cin:(t + 1) * cin, :] = slab


def _conv_stats_kernel(x_ref, w1_ref, y_ref, stats_ref, ext_ref, im2_ref,
                       *, W, P, pad):
    _build_im2col(x_ref, ext_ref, im2_ref, W=W, P=P, pad=pad)
    y = jnp.dot(w1_ref[...], im2_ref[...], preferred_element_type=jnp.float32)
    stats_ref[0, :, 0:1] = jnp.sum(y, axis=1, keepdims=True)
    stats_ref[0, :, 1:2] = jnp.sum(y * y, axis=1, keepdims=True)
    y_ref[0] = y.astype(jnp.bfloat16)


def _apply_kernel(y_ref, x_ref, scale_ref, bias_ref, w2t_ref, out_ref,
                  att_ref):
    y = y_ref[0].astype(jnp.float32)
    yb = jnp.maximum(y * scale_ref[...] + bias_ref[...], 0.0)
    z = jnp.dot(w2t_ref[...], yb.astype(jnp.bfloat16),
                preferred_element_type=jnp.float32)
    att = jax.nn.sigmoid(jnp.maximum(z, 0.0))
    out_ref[0] = x_ref[0] * (att + 1.0)
    att_ref[0] = att


def kernel(x_nchw, w1, gamma, beta, w2):
    N, Cin, H, W = x_nchw.shape
    Cout = w1.shape[-1]
    HW = H * W
    pad = _round_up(W + 1, 128)

    x_flat = x_nchw.astype(jnp.float32).reshape(N, Cin, HW)
    w1_flat = jnp.transpose(w1, (3, 0, 1, 2)).reshape(
        Cout, 9 * Cin).astype(jnp.bfloat16)
    w2t = jnp.transpose(w2, (1, 0)).astype(jnp.bfloat16)

    x_spec = pl.BlockSpec((1, Cin, HW), lambda n: (n, 0, 0))
    y_spec = pl.BlockSpec((1, Cout, HW), lambda n: (n, 0, 0))
    stats_spec = pl.BlockSpec((1, Cout, 2), lambda n: (n, 0, 0))
    w1_spec = pl.BlockSpec((Cout, 9 * Cin), lambda n: (0, 0))
    vec_spec = pl.BlockSpec((Cout, 1), lambda n: (0, 0))
    w2_spec = pl.BlockSpec((Cin, Cout), lambda n: (0, 0))

    y_bf16, stats = pl.pallas_call(
        functools.partial(_conv_stats_kernel, W=W, P=HW, pad=pad),
        grid=(N,),
        in_specs=[x_spec, w1_spec],
        out_specs=(y_spec, stats_spec),
        out_shape=(jax.ShapeDtypeStruct((N, Cout, HW), jnp.bfloat16),
                   jax.ShapeDtypeStruct((N, Cout, 2), jnp.float32)),
        scratch_shapes=[pltpu.VMEM((Cin, HW + 2 * pad), jnp.bfloat16),
                        pltpu.VMEM((9 * Cin, HW), jnp.bfloat16)],
        compiler_params=pltpu.CompilerParams(
            dimension_semantics=("parallel",)),
    )(x_flat, w1_flat)

    n_pix = jnp.float32(N * HW)
    mean = jnp.sum(stats[:, :, 0], axis=0) / n_pix
    var = jnp.sum(stats[:, :, 1], axis=0) / n_pix - mean * mean
    inv_std = jax.lax.rsqrt(var + _BN_EPS)
    g32 = gamma.astype(jnp.float32)
    scale = (g32 * inv_std).reshape(Cout, 1)
    bias = (beta.astype(jnp.float32) - mean * g32 * inv_std).reshape(Cout, 1)

    out_flat, att_flat = pl.pallas_call(
        _apply_kernel,
        grid=(N,),
        in_specs=[y_spec, x_spec, vec_spec, vec_spec, w2_spec],
        out_specs=(x_spec, x_spec),
        out_shape=(jax.ShapeDtypeStruct((N, Cin, HW), jnp.float32),
                   jax.ShapeDtypeStruct((N, Cin, HW), jnp.float32)),
        compiler_params=pltpu.CompilerParams(
            dimension_semantics=("parallel",)),
    )(y_bf16, x_flat, scale, bias, w2t)

    return out_flat.reshape(N, Cin, H, W), att_flat.reshape(N, Cin, H, W)
